# baseline (speedup 1.0000x reference)
import functools

import jax
import jax.numpy as jnp
from jax import lax
from jax.experimental import pallas as pl
from jax.experimental.pallas import tpu as pltpu
from jax.experimental.pallas import tpu_sc as plsc

_D = 32
_B = 16384
_PK = 128
_NC = 2
_NS = 16
_NW = _NC * _NS
_BPW = _B // _NW
_CHUNK = 128
_NCHUNK = _BPW // _CHUNK


def _sc_gather_build():
    mesh = plsc.VectorSubcoreMesh(core_axis_name="c", subcore_axis_name="s")

    @functools.partial(
        pl.kernel,
        mesh=mesh,
        out_type=jax.ShapeDtypeStruct((_B, _PK), jnp.float32),
        scratch_types=[
            pltpu.VMEM((_NCHUNK, _CHUNK), jnp.int32),
            pltpu.VMEM((_BPW, _PK), jnp.float32),
            pltpu.SemaphoreType.DMA,
        ],
    )
    def sc_gather(tpack_hbm, idx_hbm, out_hbm, idx_v, rows_v, sem):
        wid = lax.axis_index("s") * _NC + lax.axis_index("c")
        base = wid * _BPW
        pltpu.sync_copy(idx_hbm.at[wid], idx_v)
        copies = []
        for j in range(_NCHUNK):
            copies.append(
                pltpu.make_async_copy(
                    tpack_hbm.at[idx_v.at[j]],
                    rows_v.at[pl.ds(j * _CHUNK, _CHUNK)],
                    sem,
                )
            )
        for c in copies:
            c.start()
        for c in copies:
            c.wait()
        pltpu.sync_copy(rows_v, out_hbm.at[pl.ds(base, _BPW)])

    return sc_gather


_sc_gather = _sc_gather_build()


_ROWS_BLK = 2048


def _tc_body(x_ref, oh_ref, g_ref, bt_ref, w1t_ref, b1_ref, w2t_ref, b2_ref, o_ref):
    xm = x_ref[...] * oh_ref[...]
    x = (xm[:, 0:32] + xm[:, 32:64]) + (xm[:, 64:96] + xm[:, 96:128])
    mean = jnp.mean(x, axis=-1, keepdims=True)
    var = jnp.mean((x - mean) ** 2, axis=-1, keepdims=True)
    xhat = (x - mean) * lax.rsqrt(var + 1e-5)
    xhat = xhat * g_ref[...] + bt_ref[...]
    h = jnp.dot(xhat, w1t_ref[...], preferred_element_type=jnp.float32)
    h = h + b1_ref[...]
    h = h * jax.nn.sigmoid(h)
    o = jnp.dot(h, w2t_ref[...], preferred_element_type=jnp.float32)
    o_ref[...] = o + b2_ref[...]


@jax.jit
def _tc_mlp(x, oh, ln_gamma, ln_beta, W1t, b1, W2t, b2):
    grid = (_B // _ROWS_BLK,)
    row_spec = pl.BlockSpec((_ROWS_BLK, _PK), lambda i: (i, 0))
    out_spec = pl.BlockSpec((_ROWS_BLK, _D), lambda i: (i, 0))
    full = lambda shape: pl.BlockSpec(shape, lambda i: (0,) * len(shape))
    return pl.pallas_call(
        _tc_body,
        grid=grid,
        in_specs=[
            row_spec,
            row_spec,
            full((1, _D)),
            full((1, _D)),
            full((_D, _D)),
            full((1, _D)),
            full((_D, _D)),
            full((1, _D)),
        ],
        out_specs=out_spec,
        out_shape=jax.ShapeDtypeStruct((_B, _D), jnp.float32),
    )(x, oh, ln_gamma, ln_beta, W1t, b1, W2t, b2)


def kernel(labels, table, ln_gamma, ln_beta, W1, b1, W2, b2):
    labels = labels.astype(jnp.int32)
    one = b1[0] * 0.0 + 1.0
    tpack = table.reshape(250000, _PK) * one
    qidx = (labels >> 2).reshape(_NW, _NCHUNK, _CHUNK)
    gathered = _sc_gather(tpack, qidx)
    quarter = (labels & 3).reshape(_B, 1)
    oh = (jax.lax.broadcasted_iota(jnp.int32, (1, _PK), 1) >> 5 == quarter).astype(jnp.float32)
    return _tc_mlp(
        gathered,
        oh,
        ln_gamma.reshape(1, _D),
        ln_beta.reshape(1, _D),
        W1.T,
        b1.reshape(1, _D),
        W2.T,
        b2.reshape(1, _D),
    )


# speedup vs baseline: 1.1643x; 1.1643x over previous
import functools

import jax
import jax.numpy as jnp
from jax import lax
from jax.experimental import pallas as pl
from jax.experimental.pallas import tpu as pltpu
from jax.experimental.pallas import tpu_sc as plsc

_NUM_CLASSES = 1000000
_D = 32
_B = 16384

_NC = 2
_NS = 16
_NW = _NC * _NS
_BPW = _B // _NW
_CHUNK = 128
_NCHUNK = _BPW // _CHUNK


def _sc_gather_build():
    mesh = plsc.VectorSubcoreMesh(core_axis_name="c", subcore_axis_name="s")

    @functools.partial(
        pl.kernel,
        mesh=mesh,
        out_type=jax.ShapeDtypeStruct((_B, _D), jnp.float32),
        scratch_types=[
            pltpu.VMEM((_NCHUNK, _CHUNK), jnp.int32),
            pltpu.VMEM((_BPW, _D), jnp.float32),
            pltpu.SemaphoreType.DMA,
        ],
        compiler_params=pltpu.CompilerParams(use_tc_tiling_on_sc=False),
    )
    def sc_gather(table_hbm, idx_hbm, out_hbm, idx_v, rows_v, sem):
        wid = lax.axis_index("s") * _NC + lax.axis_index("c")
        base = wid * _BPW
        pltpu.sync_copy(idx_hbm.at[wid], idx_v)
        copies = []
        for j in range(_NCHUNK):
            copies.append(
                pltpu.make_async_copy(
                    table_hbm.at[idx_v.at[j]],
                    rows_v.at[pl.ds(j * _CHUNK, _CHUNK)],
                    sem,
                )
            )
        for c in copies:
            c.start()
        for c in copies:
            c.wait()
        pltpu.sync_copy(rows_v, out_hbm.at[pl.ds(base, _BPW)])

    return sc_gather


_sc_gather = _sc_gather_build()


_ROWS_BLK = 2048


def _tc_mlp_body(x_ref, g_ref, bt_ref, w1t_ref, b1_ref, w2t_ref, b2_ref, o_ref):
    x = x_ref[...]
    mean = jnp.mean(x, axis=-1, keepdims=True)
    var = jnp.mean((x - mean) ** 2, axis=-1, keepdims=True)
    xhat = (x - mean) * lax.rsqrt(var + 1e-5)
    xhat = xhat * g_ref[...] + bt_ref[...]
    h = jnp.dot(xhat, w1t_ref[...], preferred_element_type=jnp.float32)
    h = h + b1_ref[...]
    h = h * jax.nn.sigmoid(h)
    o = jnp.dot(h, w2t_ref[...], preferred_element_type=jnp.float32)
    o_ref[...] = o + b2_ref[...]


@jax.jit
def _tc_mlp(x, ln_gamma, ln_beta, W1t, b1, W2t, b2):
    grid = (_B // _ROWS_BLK,)
    row_spec = pl.BlockSpec((_ROWS_BLK, _D), lambda i: (i, 0))
    full = lambda shape: pl.BlockSpec(shape, lambda i: (0,) * len(shape))
    return pl.pallas_call(
        _tc_mlp_body,
        grid=grid,
        in_specs=[
            row_spec,
            full((1, _D)),
            full((1, _D)),
            full((_D, _D)),
            full((1, _D)),
            full((_D, _D)),
            full((1, _D)),
        ],
        out_specs=row_spec,
        out_shape=jax.ShapeDtypeStruct((_B, _D), jnp.float32),
    )(x, ln_gamma, ln_beta, W1t, b1, W2t, b2)


def kernel(labels, table, ln_gamma, ln_beta, W1, b1, W2, b2):
    idx = labels.reshape(_NW, _NCHUNK, _CHUNK).astype(jnp.int32)
    gathered = _sc_gather(table, idx)
    return _tc_mlp(
        gathered,
        ln_gamma.reshape(1, _D),
        ln_beta.reshape(1, _D),
        W1.T,
        b1.reshape(1, _D),
        W2.T,
        b2.reshape(1, _D),
    )
